# baseline (speedup 1.0000x reference)
import functools

import jax
import jax.numpy as jnp
from jax.experimental import pallas as pl
from jax.experimental.pallas import tpu as pltpu

_I = 64
_HI = jax.lax.Precision.HIGHEST


def _mm(a, w, precision=None):
    return jax.lax.dot_general(a, w, (((a.ndim - 1,), (0,)), ((), ())),
                               preferred_element_type=jnp.float32,
                               precision=precision)


def _mlp(h, params, precision=None):
    n = len(params)
    for k, (w, b) in enumerate(params):
        h = _mm(h, w, precision) + b
        if k < n - 1:
            h = jnp.maximum(h, 0.0)
    return h


def _bd2(w):
    z = jnp.zeros_like(w)
    return jnp.concatenate(
        [jnp.concatenate([w, z], axis=1), jnp.concatenate([z, w], axis=1)],
        axis=0)


def _gnn_kernel(x_ref, x2_ref, p_ref, pe_ref, po_ref, *refs,
                n_enc, n_postgnn, n_local, n_post):
    total_pairs = 2 * n_enc + 4 + n_postgnn + n_local + n_post
    flat = refs[:2 * total_pairs]
    o_ref = refs[2 * total_pairs]
    vals = [r[...] for r in flat]
    pairs = [(vals[2 * k], vals[2 * k + 1]) for k in range(total_pairs)]
    k0 = 0
    enc_p = pairs[k0:k0 + n_enc]; k0 += n_enc
    enc2_p = pairs[k0:k0 + n_enc]; k0 += n_enc
    (w1, _b1u), (w12, b1t), (w22, b2t), (w3, b3) = pairs[k0:k0 + 4]; k0 += 4
    pg_p = pairs[k0:k0 + n_postgnn]; k0 += n_postgnn
    loc_p = pairs[k0:k0 + n_local]; k0 += n_local
    post_p = pairs[k0:k0 + n_post]

    x2 = x2_ref[0]
    pe = pe_ref[0]
    po = po_ref[0]
    NH = x2.shape[0]
    I = _I
    n_blk = (2 * NH) // I

    enc2 = _mlp(x2, enc2_p)
    u2 = _mm(enc2, w12)

    def body(ib, _):
        i0 = ib * I
        x_i = x_ref[0, pl.ds(i0, I), :]
        p_i = p_ref[0, pl.ds(i0, I), :]
        enc_i = _mlp(x_i, enc_p)
        u_i = _mm(enc_i, w1)
        u_i2 = jnp.tile(u_i, (1, 2))

        dxe = p_i[:, 0:1] - pe[0:1, :]
        dye = p_i[:, 1:2] - pe[1:2, :]
        mfe = jnp.where(dxe * dxe + dye * dye < 1.0, 1.0, 0.0)
        dxo = p_i[:, 0:1] - po[0:1, :]
        dyo = p_i[:, 1:2] - po[1:2, :]
        mfo = jnp.where(dxo * dxo + dyo * dyo < 1.0, 1.0, 0.0)

        h1 = jnp.maximum((u2[None, :, :] - u_i2[:, None, :]) + b1t, 0.0)
        h2 = jnp.maximum(_mm(h1.reshape(I * NH, 128), w22) + b2t, 0.0)

        mfc = jnp.concatenate(
            [jnp.broadcast_to(mfe[:, :, None], (I, NH, 64)),
             jnp.broadcast_to(mfo[:, :, None], (I, NH, 64))], axis=2)
        a3 = jnp.sum(h2.reshape(I, NH, 128) * mfc, axis=1)
        agg = a3[:, :64] + a3[:, 64:]
        deg = (jnp.sum(mfe, axis=1, keepdims=True)
               + jnp.sum(mfo, axis=1, keepdims=True)) - 1.0

        s2b = jnp.maximum(_mm(jnp.maximum(b1t, 0.0), w22) + b2t, 0.0)
        gnn_out = _mm(agg - s2b[:, :64], w3) + deg * b3

        post_gnn = _mlp(gnn_out, pg_p)
        local = _mlp(x_i, loc_p)
        o_ref[0, pl.ds(i0, I), :] = _mlp(post_gnn + local, post_p)
        return 0

    jax.lax.fori_loop(0, n_blk, body, 0, unroll=True)


def kernel(x, p, comm_radius, enc_params, gnn_params, post_gnn_params,
           local_params, post_params):
    B, N, F = x.shape
    I = _I
    NH = N // 2
    p_scaled = p / jnp.asarray(comm_radius, jnp.float32)
    pt = jnp.swapaxes(p_scaled, 1, 2)
    pe = pt[:, :, :NH]
    po = pt[:, :, NH:]
    x2 = jnp.concatenate([x[:, :NH, :], x[:, NH:, :]], axis=2)

    (gw1, gb1), (gw2, gb2), (gw3, gb3) = gnn_params

    weight_arrays = []
    for w, b in enc_params:
        weight_arrays += [w, b.reshape(1, -1)]
    for w, b in enc_params:
        weight_arrays += [_bd2(w), jnp.tile(b.reshape(1, -1), (1, 2))]
    weight_arrays += [gw1, gb1.reshape(1, -1)]
    weight_arrays += [_bd2(gw1), jnp.tile(gb1.reshape(1, -1), (1, 2))]
    weight_arrays += [_bd2(gw2), jnp.tile(gb2.reshape(1, -1), (1, 2))]
    weight_arrays += [gw3, gb3.reshape(1, -1)]
    for group in (post_gnn_params, local_params, post_params):
        for w, b in group:
            weight_arrays += [w, b.reshape(1, -1)]

    grid = (B,)
    in_specs = [
        pl.BlockSpec((1, N, F), lambda b: (b, 0, 0)),
        pl.BlockSpec((1, NH, 2 * F), lambda b: (b, 0, 0)),
        pl.BlockSpec((1, N, p.shape[2]), lambda b: (b, 0, 0)),
        pl.BlockSpec((1, 2, NH), lambda b: (b, 0, 0)),
        pl.BlockSpec((1, 2, NH), lambda b: (b, 0, 0)),
    ] + [pl.BlockSpec(w.shape, lambda b: (0,) * w.ndim) for w in weight_arrays]

    out = pl.pallas_call(
        functools.partial(_gnn_kernel, n_enc=len(enc_params),
                          n_postgnn=len(post_gnn_params),
                          n_local=len(local_params), n_post=len(post_params)),
        grid=grid,
        in_specs=in_specs,
        out_specs=pl.BlockSpec((1, N, 32), lambda b: (b, 0, 0)),
        out_shape=jax.ShapeDtypeStruct((B, N, 32), jnp.float32),
    )(x, x2, p_scaled, pe, po, *weight_arrays)
    return out


# speedup vs baseline: 1.2024x; 1.2024x over previous
import functools

import jax
import jax.numpy as jnp
from jax.experimental import pallas as pl
from jax.experimental.pallas import tpu as pltpu

_I = 64


def _mm(a, w, precision=None):
    return jax.lax.dot_general(a, w, (((a.ndim - 1,), (0,)), ((), ())),
                               preferred_element_type=jnp.float32,
                               precision=precision)


def _mlp(h, params, precision=None):
    n = len(params)
    for k, (w, b) in enumerate(params):
        h = _mm(h, w, precision) + b
        if k < n - 1:
            h = jnp.maximum(h, 0.0)
    return h


def _bd2(w):
    z = jnp.zeros_like(w)
    return jnp.concatenate(
        [jnp.concatenate([w, z], axis=1), jnp.concatenate([z, w], axis=1)],
        axis=0)


def _t2(b):
    return jnp.tile(b, (1, 2))


def _gnn_kernel(x_ref, p_ref, r2_ref, *refs,
                n_enc, n_postgnn, n_local, n_post):
    total_pairs = n_enc + 3 + n_postgnn + n_local + n_post
    flat = refs[:2 * total_pairs]
    o_ref = refs[2 * total_pairs]
    vals = [r[...] for r in flat]
    pairs = [(vals[2 * k], vals[2 * k + 1]) for k in range(total_pairs)]
    k0 = 0
    enc_p = pairs[k0:k0 + n_enc]; k0 += n_enc
    (w1, b1), (w2, b2), (w3, b3) = pairs[k0:k0 + 3]; k0 += 3
    pg_p = pairs[k0:k0 + n_postgnn]; k0 += n_postgnn
    loc_p = pairs[k0:k0 + n_local]; k0 += n_local
    post_p = pairs[k0:k0 + n_post]

    x = x_ref[0]
    p = p_ref[0]
    r2 = r2_ref[0, 0]
    N = x.shape[0]
    NH = N // 2
    I = _I
    n_blk = N // I

    enc2_p = [(_bd2(w), _t2(b)) for (w, b) in enc_p]
    w12, b1t = _bd2(w1), _t2(b1)
    w22, b2t = _bd2(w2), _t2(b2)

    x2 = jnp.concatenate([x[:NH, :], x[NH:, :]], axis=1)

    ones_r = jnp.ones((1, 128), jnp.float32)
    hi = jax.lax.Precision.HIGHEST
    px_b = _mm(p[:, 0:1], ones_r, hi)
    py_b = _mm(p[:, 1:2], ones_r, hi)

    enc2 = _mlp(x2, enc2_p)
    u2 = _mm(enc2, w12)

    lane = jax.lax.broadcasted_iota(jnp.int32, (1, 128), 1)
    sel_e = (lane < 64).astype(jnp.float32)
    sel_o = (lane >= 64).astype(jnp.float32)
    pjx_b = px_b[:NH] * sel_e + px_b[NH:] * sel_o
    pjy_b = py_b[:NH] * sel_e + py_b[NH:] * sel_o

    s2b = jnp.maximum(_mm(jnp.maximum(b1t, 0.0), w22) + b2t, 0.0)

    for ib in range(n_blk):
        i0 = ib * I
        x_i = x_ref[0, pl.ds(i0, I), :]
        enc_i = _mlp(x_i, enc_p)
        u_i = _mm(enc_i, w1)
        u_i2 = jnp.tile(u_i, (1, 2))

        pxi_b = px_b[i0:i0 + I]
        pyi_b = py_b[i0:i0 + I]

        dx = pxi_b[:, None, :] - pjx_b[None, :, :]
        dy = pyi_b[:, None, :] - pjy_b[None, :, :]
        rm = jnp.where(dx * dx + dy * dy < r2, 1.0, 0.0)

        h1 = jnp.maximum((u2[None, :, :] - u_i2[:, None, :]) + b1t, 0.0)
        h2 = jnp.maximum(_mm(h1.reshape(I * NH, 128), w22) + b2t, 0.0)

        a3 = jnp.sum(h2.reshape(I, NH, 128) * rm, axis=1)
        agg = a3[:, :64] + a3[:, 64:]
        deg3 = jnp.sum(rm, axis=1)
        deg = deg3[:, :64] + deg3[:, 64:] - 1.0

        gnn_out = _mm(agg - s2b[:, :64], w3) + deg * b3

        post_gnn = _mlp(gnn_out, pg_p)
        local = _mlp(x_i, loc_p)
        o_ref[0, pl.ds(i0, I), :] = _mlp(post_gnn + local, post_p)


def kernel(x, p, comm_radius, enc_params, gnn_params, post_gnn_params,
           local_params, post_params):
    B, N, F = x.shape
    r2 = jnp.asarray(comm_radius, jnp.float32).reshape(1, 1) ** 2

    weight_arrays = []
    for group in (enc_params, gnn_params, post_gnn_params, local_params,
                  post_params):
        for w, b in group:
            weight_arrays += [w, b.reshape(1, -1)]

    grid = (B,)
    in_specs = [
        pl.BlockSpec((1, N, F), lambda b: (b, 0, 0)),
        pl.BlockSpec((1, N, p.shape[2]), lambda b: (b, 0, 0)),
        pl.BlockSpec((1, 1), lambda b: (0, 0)),
    ] + [pl.BlockSpec(w.shape, lambda b: (0,) * w.ndim) for w in weight_arrays]

    out = pl.pallas_call(
        functools.partial(_gnn_kernel, n_enc=len(enc_params),
                          n_postgnn=len(post_gnn_params),
                          n_local=len(local_params), n_post=len(post_params)),
        grid=grid,
        in_specs=in_specs,
        out_specs=pl.BlockSpec((1, N, 32), lambda b: (b, 0, 0)),
        out_shape=jax.ShapeDtypeStruct((B, N, 32), jnp.float32),
    )(x, p, r2, *weight_arrays)
    return out


# speedup vs baseline: 1.2065x; 1.0034x over previous
import functools

import jax
import jax.numpy as jnp
from jax.experimental import pallas as pl
from jax.experimental.pallas import tpu as pltpu

_I = 64


def _mm(a, w, precision=None):
    return jax.lax.dot_general(a, w, (((a.ndim - 1,), (0,)), ((), ())),
                               preferred_element_type=jnp.float32,
                               precision=precision)


def _mlp(h, params, precision=None):
    n = len(params)
    for k, (w, b) in enumerate(params):
        h = _mm(h, w, precision) + b
        if k < n - 1:
            h = jnp.maximum(h, 0.0)
    return h


def _bd2(w):
    z = jnp.zeros_like(w)
    return jnp.concatenate(
        [jnp.concatenate([w, z], axis=1), jnp.concatenate([z, w], axis=1)],
        axis=0)


def _t2(b):
    return jnp.tile(b, (1, 2))


def _gnn_kernel(x_ref, p_ref, r2_ref, *refs,
                n_enc, n_postgnn, n_local, n_post):
    total_pairs = n_enc + 3 + n_postgnn + n_local + n_post
    flat = refs[:2 * total_pairs]
    o_ref = refs[2 * total_pairs]
    vals = [r[...] for r in flat]
    pairs = [(vals[2 * k], vals[2 * k + 1]) for k in range(total_pairs)]
    k0 = 0
    enc_p = pairs[k0:k0 + n_enc]; k0 += n_enc
    (w1, b1), (w2, b2), (w3, b3) = pairs[k0:k0 + 3]; k0 += 3
    pg_p = pairs[k0:k0 + n_postgnn]; k0 += n_postgnn
    loc_p = pairs[k0:k0 + n_local]; k0 += n_local
    post_p = pairs[k0:k0 + n_post]

    x = x_ref[0]
    p = p_ref[0]
    r2 = r2_ref[0, 0]
    N = x.shape[0]
    NH = N // 2
    I = _I
    n_blk = N // I

    enc2_p = [(_bd2(w), _t2(b)) for (w, b) in enc_p]
    w12, b1t = _bd2(w1), _t2(b1)
    w22, b2t = _bd2(w2), _t2(b2)

    x2 = jnp.concatenate([x[:NH, :], x[NH:, :]], axis=1)

    ones_r = jnp.ones((1, 128), jnp.float32)
    hi = jax.lax.Precision.HIGHEST
    px_b = _mm(p[:, 0:1], ones_r, hi)
    py_b = _mm(p[:, 1:2], ones_r, hi)

    enc2 = _mlp(x2, enc2_p)
    u2 = _mm(enc2, w12)

    lane = jax.lax.broadcasted_iota(jnp.int32, (1, 128), 1)
    sel_e = (lane < 64).astype(jnp.float32)
    sel_o = (lane >= 64).astype(jnp.float32)
    pjx_b = px_b[:NH] * sel_e + px_b[NH:] * sel_o
    pjy_b = py_b[:NH] * sel_e + py_b[NH:] * sel_o

    s2b = jnp.maximum(_mm(jnp.maximum(b1t, 0.0), w22) + b2t, 0.0)

    for ib in range(n_blk):
        i0 = ib * I
        x_i = x_ref[0, pl.ds(i0, I), :]
        enc_i = _mlp(x_i, enc_p)
        u_i = _mm(enc_i, w1)
        u_i2 = jnp.tile(u_i, (1, 2)) - b1t

        pxi_b = px_b[i0:i0 + I]
        pyi_b = py_b[i0:i0 + I]

        dx = pxi_b[:, None, :] - pjx_b[None, :, :]
        dy = pyi_b[:, None, :] - pjy_b[None, :, :]
        rm = jnp.where(dx * dx + dy * dy < r2, 1.0, 0.0)

        h1 = jnp.maximum(u2[None, :, :] - u_i2[:, None, :], 0.0)
        h2 = jnp.maximum(_mm(h1.reshape(I * NH, 128), w22) + b2t, 0.0)

        a3 = jnp.sum(h2.reshape(I, NH, 128) * rm, axis=1)
        agg = a3[:, :64] + a3[:, 64:]
        deg3 = jnp.sum(rm, axis=1)
        deg = deg3[:, :64] + deg3[:, 64:] - 1.0

        gnn_out = _mm(agg - s2b[:, :64], w3) + deg * b3

        post_gnn = _mlp(gnn_out, pg_p)
        local = _mlp(x_i, loc_p)
        o_ref[0, pl.ds(i0, I), :] = _mlp(post_gnn + local, post_p)


def kernel(x, p, comm_radius, enc_params, gnn_params, post_gnn_params,
           local_params, post_params):
    B, N, F = x.shape
    r2 = jnp.asarray(comm_radius, jnp.float32).reshape(1, 1) ** 2

    weight_arrays = []
    for group in (enc_params, gnn_params, post_gnn_params, local_params,
                  post_params):
        for w, b in group:
            weight_arrays += [w, b.reshape(1, -1)]

    grid = (B,)
    in_specs = [
        pl.BlockSpec((1, N, F), lambda b: (b, 0, 0)),
        pl.BlockSpec((1, N, p.shape[2]), lambda b: (b, 0, 0)),
        pl.BlockSpec((1, 1), lambda b: (0, 0)),
    ] + [pl.BlockSpec(w.shape, lambda b: (0,) * w.ndim) for w in weight_arrays]

    out = pl.pallas_call(
        functools.partial(_gnn_kernel, n_enc=len(enc_params),
                          n_postgnn=len(post_gnn_params),
                          n_local=len(local_params), n_post=len(post_params)),
        grid=grid,
        in_specs=in_specs,
        out_specs=pl.BlockSpec((1, N, 32), lambda b: (b, 0, 0)),
        out_shape=jax.ShapeDtypeStruct((B, N, 32), jnp.float32),
        compiler_params=pltpu.CompilerParams(
            dimension_semantics=("parallel",)),
    )(x, p, r2, *weight_arrays)
    return out
